# baseline (speedup 1.0000x reference)
import functools

import jax
import jax.numpy as jnp
from jax import lax
from jax.experimental import pallas as pl
from jax.experimental.pallas import tpu as pltpu
from jax.experimental.pallas import tpu_sc as plsc

VOCAB = 100000
EMBED_DIM = 32
NUM_CLASSES = 2
BATCH = 16384
SEQ = 50

LANES = 16
NUM_WORKERS = 32
B_PER_W = BATCH // NUM_WORKERS
GROUPS = B_PER_W // LANES


def _project_kernel(tbl_ref, wt_ref, b_ref, out_ref):
    p = lax.dot_general(
        wt_ref[...],
        tbl_ref[...],
        (((1,), (1,)), ((), ())),
        preferred_element_type=jnp.float32,
    )
    pb = ((p + b_ref[...]) * (1.0 / SEQ)).astype(jnp.bfloat16)
    bits = lax.bitcast_convert_type(pb, jnp.uint16)
    lo = bits[0:1, :].astype(jnp.uint32)
    hi = bits[1:2, :].astype(jnp.uint32)
    packed = lax.bitcast_convert_type(lo | (hi << 16), jnp.int32)
    blk = tbl_ref.shape[0]
    start = pl.multiple_of(pl.program_id(0) * blk, 128)
    out_ref[pl.ds(start, blk)] = packed.reshape((blk,))


def _sc_lookup(p_hbm, text_hbm, out_hbm, p_v, text_v, out_v, sem_p, sem_t):
    wid = lax.axis_index("s") * 2 + lax.axis_index("c")
    cp_p = pltpu.async_copy(p_hbm, p_v, sem_p)
    cp_t = pltpu.async_copy(text_hbm.at[wid], text_v, sem_t)
    cp_t.wait()
    cp_p.wait()

    lane = lax.iota(jnp.int32, LANES)
    col0 = jnp.zeros((LANES,), jnp.int32)
    col1 = jnp.ones((LANES,), jnp.int32)
    hi_mask = jnp.full((LANES,), -65536, jnp.int32)
    shift16 = jnp.full((LANES,), 16, jnp.int32)

    unroll = 10

    def group_body(g, carry):
        def s_body(c, acc):
            a0, a1 = acc
            for k in range(unroll):
                idx = text_v[g, c * unroll + k, :]
                w = plsc.load_gather(p_v, [idx])
                lo = plsc.bitcast(lax.shift_left(w, shift16), jnp.float32)
                hi = plsc.bitcast(lax.bitwise_and(w, hi_mask), jnp.float32)
                a0 = a0 + lo
                a1 = a1 + hi
            return a0, a1

        zero = jnp.zeros((LANES,), jnp.float32)
        a0, a1 = lax.fori_loop(0, SEQ // unroll, s_body, (zero, zero))
        rows = g * LANES + lane
        plsc.store_scatter(out_v, [rows, col0], a0)
        plsc.store_scatter(out_v, [rows, col1], a1)
        return carry

    lax.fori_loop(0, GROUPS, group_body, 0)
    pltpu.sync_copy(out_v, out_hbm.at[pl.ds(wid * B_PER_W, B_PER_W)])


def kernel(text, embedding_table, fc_weight, fc_bias):
    blk = 25000
    packed = pl.pallas_call(
        _project_kernel,
        grid=(VOCAB // blk,),
        in_specs=[
            pl.BlockSpec((blk, EMBED_DIM), lambda i: (i, 0)),
            pl.BlockSpec((NUM_CLASSES, EMBED_DIM), lambda i: (0, 0)),
            pl.BlockSpec((NUM_CLASSES, 1), lambda i: (0, 0)),
        ],
        out_specs=pl.BlockSpec((VOCAB,), lambda i: (0,)),
        out_shape=jax.ShapeDtypeStruct((VOCAB,), jnp.int32),
    )(
        embedding_table,
        fc_weight.astype(jnp.float32).T,
        fc_bias.astype(jnp.float32).reshape(NUM_CLASSES, 1),
    )

    text_r = (
        text.astype(jnp.int32)
        .T.reshape(SEQ, NUM_WORKERS, GROUPS, LANES)
        .transpose(1, 2, 0, 3)
    )

    mesh = plsc.VectorSubcoreMesh(core_axis_name="c", subcore_axis_name="s")
    sc = pl.kernel(
        _sc_lookup,
        mesh=mesh,
        out_type=jax.ShapeDtypeStruct((BATCH, NUM_CLASSES), jnp.float32),
        scratch_types=[
            pltpu.VMEM((VOCAB,), jnp.int32),
            pltpu.VMEM((GROUPS, SEQ, LANES), jnp.int32),
            pltpu.VMEM((B_PER_W, NUM_CLASSES), jnp.float32),
            pltpu.SemaphoreType.DMA,
            pltpu.SemaphoreType.DMA,
        ],
        compiler_params=pltpu.CompilerParams(
            needs_layout_passes=False,
            use_tc_tiling_on_sc=False,
        ),
    )
    return sc(packed, text_r)


# speedup vs baseline: 1.3471x; 1.3471x over previous
import functools

import jax
import jax.numpy as jnp
from jax import lax
from jax.experimental import pallas as pl
from jax.experimental.pallas import tpu as pltpu
from jax.experimental.pallas import tpu_sc as plsc

VOCAB = 100000
EMBED_DIM = 32
NUM_CLASSES = 2
BATCH = 16384
SEQ = 50

LANES = 16
NUM_WORKERS = 32
B_PER_W = BATCH // NUM_WORKERS
GROUPS = B_PER_W // LANES


def _project_kernel(tbl_ref, wt_ref, b_ref, out_ref):
    p = lax.dot_general(
        wt_ref[...],
        tbl_ref[...],
        (((1,), (1,)), ((), ())),
        preferred_element_type=jnp.float32,
    )
    pb = ((p + b_ref[...]) * (1.0 / SEQ)).astype(jnp.bfloat16)
    bits = lax.bitcast_convert_type(pb, jnp.uint16)
    lo = bits[0:1, :].astype(jnp.uint32)
    hi = bits[1:2, :].astype(jnp.uint32)
    packed = lax.bitcast_convert_type(lo | (hi << 16), jnp.int32)
    blk = tbl_ref.shape[0]
    start = pl.multiple_of(pl.program_id(0) * blk, 128)
    out_ref[pl.ds(start, blk)] = packed.reshape((blk,))


def _sc_lookup(p_hbm, text_hbm, out_hbm, p_v, text_v, out_v, sem_p, sem_t):
    wid = lax.axis_index("s") * 2 + lax.axis_index("c")
    cp_p = pltpu.async_copy(p_hbm, p_v, sem_p)
    cp_t = pltpu.async_copy(text_hbm.at[wid], text_v, sem_t)
    cp_t.wait()
    cp_p.wait()

    lane = lax.iota(jnp.int32, LANES)
    col0 = jnp.zeros((LANES,), jnp.int32)
    col1 = jnp.ones((LANES,), jnp.int32)
    hi_mask = jnp.full((LANES,), -65536, jnp.int32)
    shift16 = jnp.full((LANES,), 16, jnp.int32)

    unroll = 10

    def group_body(g, carry):
        def s_body(c, acc):
            a0, a1 = acc
            for k in range(unroll):
                idx = text_v[c * unroll + k, pl.ds(g * LANES, LANES)]
                w = plsc.load_gather(p_v, [idx])
                lo = plsc.bitcast(lax.shift_left(w, shift16), jnp.float32)
                hi = plsc.bitcast(lax.bitwise_and(w, hi_mask), jnp.float32)
                a0 = a0 + lo
                a1 = a1 + hi
            return a0, a1

        zero = jnp.zeros((LANES,), jnp.float32)
        a0, a1 = lax.fori_loop(0, SEQ // unroll, s_body, (zero, zero))
        rows = g * LANES + lane
        plsc.store_scatter(out_v, [rows, col0], a0)
        plsc.store_scatter(out_v, [rows, col1], a1)
        return carry

    lax.fori_loop(0, GROUPS, group_body, 0)
    pltpu.sync_copy(out_v, out_hbm.at[pl.ds(wid * B_PER_W, B_PER_W)])


def kernel(text, embedding_table, fc_weight, fc_bias):
    blk = 25000
    packed = pl.pallas_call(
        _project_kernel,
        grid=(VOCAB // blk,),
        in_specs=[
            pl.BlockSpec((blk, EMBED_DIM), lambda i: (i, 0)),
            pl.BlockSpec((NUM_CLASSES, EMBED_DIM), lambda i: (0, 0)),
            pl.BlockSpec((NUM_CLASSES, 1), lambda i: (0, 0)),
        ],
        out_specs=pl.BlockSpec((VOCAB,), lambda i: (0,)),
        out_shape=jax.ShapeDtypeStruct((VOCAB,), jnp.int32),
    )(
        embedding_table,
        fc_weight.astype(jnp.float32).T,
        fc_bias.astype(jnp.float32).reshape(NUM_CLASSES, 1),
    )

    text_r = (
        text.astype(jnp.int32)
        .T.reshape(SEQ, NUM_WORKERS, B_PER_W)
        .transpose(1, 0, 2)
    )

    mesh = plsc.VectorSubcoreMesh(core_axis_name="c", subcore_axis_name="s")
    sc = pl.kernel(
        _sc_lookup,
        mesh=mesh,
        out_type=jax.ShapeDtypeStruct((BATCH, NUM_CLASSES), jnp.float32),
        scratch_types=[
            pltpu.VMEM((VOCAB,), jnp.int32),
            pltpu.VMEM((SEQ, B_PER_W), jnp.int32),
            pltpu.VMEM((B_PER_W, NUM_CLASSES), jnp.float32),
            pltpu.SemaphoreType.DMA,
            pltpu.SemaphoreType.DMA,
        ],
        compiler_params=pltpu.CompilerParams(
            needs_layout_passes=False,
            use_tc_tiling_on_sc=False,
        ),
    )
    return sc(packed, text_r)


# speedup vs baseline: 1.3485x; 1.0010x over previous
import functools

import jax
import jax.numpy as jnp
from jax import lax
from jax.experimental import pallas as pl
from jax.experimental.pallas import tpu as pltpu
from jax.experimental.pallas import tpu_sc as plsc

VOCAB = 100000
EMBED_DIM = 32
NUM_CLASSES = 2
BATCH = 16384
SEQ = 50

LANES = 16
NUM_WORKERS = 32
B_PER_W = BATCH // NUM_WORKERS
GROUPS = B_PER_W // LANES


def _project_kernel(tbl_ref, wt_ref, b_ref, out_ref):
    p = lax.dot_general(
        wt_ref[...],
        tbl_ref[...],
        (((1,), (1,)), ((), ())),
        preferred_element_type=jnp.float32,
    )
    pb = ((p + b_ref[...]) * (1.0 / SEQ)).astype(jnp.bfloat16)
    bits = lax.bitcast_convert_type(pb, jnp.uint16)
    lo = bits[0:1, :].astype(jnp.uint32)
    hi = bits[1:2, :].astype(jnp.uint32)
    packed = lax.bitcast_convert_type(lo | (hi << 16), jnp.int32)
    blk = tbl_ref.shape[0]
    start = pl.multiple_of(pl.program_id(0) * blk, 128)
    out_ref[pl.ds(start, blk)] = packed.reshape((blk,))


def _sc_lookup(p_hbm, text_hbm, out_hbm, p_v, text_v, out_v, sem_p, sem_t):
    wid = lax.axis_index("s") * 2 + lax.axis_index("c")
    cp_p = pltpu.async_copy(p_hbm, p_v, sem_p)
    cp_t = pltpu.async_copy(text_hbm.at[wid], text_v, sem_t)
    cp_t.wait()
    cp_p.wait()

    lane = lax.iota(jnp.int32, LANES)
    col0 = jnp.zeros((LANES,), jnp.int32)
    col1 = jnp.ones((LANES,), jnp.int32)
    hi_mask = jnp.full((LANES,), -65536, jnp.int32)
    shift16 = jnp.full((LANES,), 16, jnp.int32)

    unroll = 5

    def group_body(g, carry):
        def s_body(c, acc):
            a0, a1 = acc
            for k in range(unroll):
                idx = text_v[c * unroll + k, pl.ds(g * LANES, LANES)]
                w = plsc.load_gather(p_v, [idx])
                lo = plsc.bitcast(lax.shift_left(w, shift16), jnp.float32)
                hi = plsc.bitcast(lax.bitwise_and(w, hi_mask), jnp.float32)
                a0 = a0 + lo
                a1 = a1 + hi
            return a0, a1

        zero = jnp.zeros((LANES,), jnp.float32)
        a0, a1 = lax.fori_loop(0, SEQ // unroll, s_body, (zero, zero))
        rows = g * LANES + lane
        plsc.store_scatter(out_v, [rows, col0], a0)
        plsc.store_scatter(out_v, [rows, col1], a1)
        return carry

    lax.fori_loop(0, GROUPS, group_body, 0)
    pltpu.sync_copy(out_v, out_hbm.at[pl.ds(wid * B_PER_W, B_PER_W)])


def kernel(text, embedding_table, fc_weight, fc_bias):
    blk = 25000
    packed = pl.pallas_call(
        _project_kernel,
        grid=(VOCAB // blk,),
        in_specs=[
            pl.BlockSpec((blk, EMBED_DIM), lambda i: (i, 0)),
            pl.BlockSpec((NUM_CLASSES, EMBED_DIM), lambda i: (0, 0)),
            pl.BlockSpec((NUM_CLASSES, 1), lambda i: (0, 0)),
        ],
        out_specs=pl.BlockSpec((VOCAB,), lambda i: (0,)),
        out_shape=jax.ShapeDtypeStruct((VOCAB,), jnp.int32),
    )(
        embedding_table,
        fc_weight.astype(jnp.float32).T,
        fc_bias.astype(jnp.float32).reshape(NUM_CLASSES, 1),
    )

    text_r = (
        text.astype(jnp.int32)
        .T.reshape(SEQ, NUM_WORKERS, B_PER_W)
        .transpose(1, 0, 2)
    )

    mesh = plsc.VectorSubcoreMesh(core_axis_name="c", subcore_axis_name="s")
    sc = pl.kernel(
        _sc_lookup,
        mesh=mesh,
        out_type=jax.ShapeDtypeStruct((BATCH, NUM_CLASSES), jnp.float32),
        scratch_types=[
            pltpu.VMEM((VOCAB,), jnp.int32),
            pltpu.VMEM((SEQ, B_PER_W), jnp.int32),
            pltpu.VMEM((B_PER_W, NUM_CLASSES), jnp.float32),
            pltpu.SemaphoreType.DMA,
            pltpu.SemaphoreType.DMA,
        ],
        compiler_params=pltpu.CompilerParams(
            needs_layout_passes=False,
            use_tc_tiling_on_sc=False,
        ),
    )
    return sc(packed, text_r)
